# baseline (speedup 1.0000x reference)
import jax
import jax.numpy as jnp
from jax.experimental import pallas as pl
from jax.experimental.pallas import tpu as pltpu

_B, _E, _S, _F = 16, 1024, 256, 8
_A, _C = 16, 8
_BBLK = 2
_BSTEPS = _B // _BBLK


def _body(scale_ref, alpha_ref, m_ref, eq_ref, x_ref, o_ref):
    m = m_ref[...]
    eq = eq_ref[...]
    scale = scale_ref[0]
    alpha = alpha_ref[0]

    w1 = jax.lax.dot_general(m, eq, (((0,), (0,)), ((), ())),
                             preferred_element_type=jnp.float32)
    w1s = w1 * (-scale)
    w2s = (jnp.sum(eq, axis=0)[None, :] - w1) * alpha

    for i in range(_BBLK):
        t = x_ref[i]
        xs = t.sum(axis=1)
        obs = jnp.logical_not(jnp.isnan(xs))
        xc = jnp.where(obs, xs, 0.0)
        obs_f = obs.astype(jnp.float32)

        part = jax.lax.dot_general(xc, w1s, (((0,), (0,)), ((), ())),
                                   preferred_element_type=jnp.float32)
        part += jax.lax.dot_general(obs_f, w2s, (((0,), (0,)), ((), ())),
                                    preferred_element_type=jnp.float32)
        o_ref[i] = part


@jax.jit
def kernel(input_seq, eq_classes, scale, alpha, edge_masks):
    xt = jnp.transpose(input_seq, (0, 1, 3, 2))
    m_f = edge_masks.astype(jnp.float32)
    grid = (_BSTEPS,)
    return pl.pallas_call(
        _body,
        grid=grid,
        in_specs=[
            pl.BlockSpec(memory_space=pltpu.SMEM),
            pl.BlockSpec(memory_space=pltpu.SMEM),
            pl.BlockSpec((_A, _E), lambda b: (0, 0)),
            pl.BlockSpec((_A, _C), lambda b: (0, 0)),
            pl.BlockSpec((_BBLK, _E, _F, _S), lambda b: (b, 0, 0, 0)),
        ],
        out_specs=pl.BlockSpec((_BBLK, _S, _C), lambda b: (b, 0, 0)),
        out_shape=jax.ShapeDtypeStruct((_B, _S, _C), jnp.float32),
        compiler_params=pltpu.CompilerParams(
            dimension_semantics=("parallel",),
        ),
    )(scale.reshape(1), alpha.reshape(1), m_f, eq_classes, xt)
